# baseline (speedup 1.0000x reference)
import functools

import jax
import jax.numpy as jnp
from jax import lax
from jax.experimental import pallas as pl
from jax.experimental.pallas import tpu as pltpu
from jax.experimental.pallas import tpu_sc as plsc

_B, _S, _E = 16384, 200, 64
_R = 128

_NC, _NS = 2, 16
_NW = _NC * _NS
_PER_W = (_B * _S) // _NW
_CHUNK = 128
_NCH = _PER_W // _CHUNK


def _rank_body(ts_ref, out_ref):
    ts = ts_ref[...]
    ib = lax.bitcast_convert_type(ts, jnp.int32)
    ikey = ib ^ ((ib >> 31) & jnp.int32(0x7FFFFFFF))
    ikey = jnp.where(ts == 0.0, 0, ikey)
    a = ikey[:, :, None]
    b = ikey[:, None, :]
    k2 = lax.broadcasted_iota(jnp.int32, (_S, _S), 1)
    j2 = lax.broadcasted_iota(jnp.int32, (_S, _S), 0)
    tri01 = (k2 < j2).astype(jnp.int32)[None, :, :]
    cmp = b < (a + tri01)
    rank = jnp.sum(cmp.astype(jnp.float32), axis=2)
    row = pl.program_id(0) * _R + lax.broadcasted_iota(jnp.int32, (_R, _S), 0)
    out_ref[...] = rank.astype(jnp.int32) + row * _S


def _dest_rows(ts):
    return pl.pallas_call(
        _rank_body,
        grid=(_B // _R,),
        in_specs=[pl.BlockSpec((_R, _S), lambda i: (i, 0))],
        out_specs=pl.BlockSpec((_R, _S), lambda i: (i, 0)),
        out_shape=jax.ShapeDtypeStruct((_B, _S), jnp.int32),
    )(ts)


def _sc_scatter_body(emb_hbm, idx_hbm, out_hbm, tbl2, idx_v, sem):
    wid = lax.axis_index("s") * _NC + lax.axis_index("c")
    base = wid * _PER_W
    pltpu.sync_copy(emb_hbm, tbl2.at[pl.ds(0, _S)])
    pltpu.sync_copy(emb_hbm, tbl2.at[pl.ds(_S, _S)])

    def body(c, carry):
        g = base + c * _CHUNK
        phase = lax.rem(c * _CHUNK, _S)
        pltpu.sync_copy(idx_hbm.at[pl.ds(g, _CHUNK)], idx_v)
        pltpu.async_copy(tbl2.at[pl.ds(phase, _CHUNK)],
                         out_hbm.at[idx_v], sem).wait()
        return carry

    lax.fori_loop(0, _NCH, body, 0)


_sc_scatter = functools.partial(
    pl.kernel,
    mesh=plsc.VectorSubcoreMesh(core_axis_name="c", subcore_axis_name="s"),
    compiler_params=pltpu.CompilerParams(use_tc_tiling_on_sc=False),
    out_type=jax.ShapeDtypeStruct((_B * _S, _E), jnp.float32),
    scratch_types=[
        pltpu.VMEM((2 * _S, _E), jnp.float32),
        pltpu.VMEM((_CHUNK,), jnp.int32),
        pltpu.SemaphoreType.DMA,
    ],
)(_sc_scatter_body)


def kernel(ts, pos_embeddings):
    dest = _dest_rows(ts).reshape(_B * _S)
    out = _sc_scatter(pos_embeddings, dest)
    return out.reshape(_B, _S, _E)


# speedup vs baseline: 1.1473x; 1.1473x over previous
import functools

import jax
import jax.numpy as jnp
from jax import lax
from jax.experimental import pallas as pl
from jax.experimental.pallas import tpu as pltpu
from jax.experimental.pallas import tpu_sc as plsc

_B, _S, _E = 16384, 200, 64
_R = 128

_NC, _NS = 2, 16
_NW = _NC * _NS
_PER_W = (_B * _S) // _NW
_CHUNK = 128
_NCH = _PER_W // _CHUNK


def _rank_body(ts_ref, out_ref):
    ts = ts_ref[...]
    ib = lax.bitcast_convert_type(ts, jnp.int32)
    ikey = ib ^ ((ib >> 31) & jnp.int32(0x7FFFFFFF))
    ikey = jnp.where(ts == 0.0, 0, ikey)
    a = ikey[:, :, None]
    b = ikey[:, None, :]
    k2 = lax.broadcasted_iota(jnp.int32, (_S, _S), 1)
    j2 = lax.broadcasted_iota(jnp.int32, (_S, _S), 0)
    tri01 = (k2 < j2).astype(jnp.int32)[None, :, :]
    cmp = b < (a + tri01)
    rank = jnp.sum(cmp.astype(jnp.float32), axis=2)
    row = pl.program_id(0) * _R + lax.broadcasted_iota(jnp.int32, (_R, _S), 0)
    dest = rank.astype(jnp.int32) + row * _S
    out_ref[...] = dest.reshape(_R * _S // 128, 128)


def _dest_rows(ts):
    return pl.pallas_call(
        _rank_body,
        grid=(_B // _R,),
        in_specs=[pl.BlockSpec((_R, _S), lambda i: (i, 0))],
        out_specs=pl.BlockSpec((_R * _S // 128, 128), lambda i: (i, 0)),
        out_shape=jax.ShapeDtypeStruct((_B * _S // 128, 128), jnp.int32),
    )(ts)


def _sc_scatter_body(emb_hbm, idx_hbm, out_hbm, tbl2, idx_all, sem):
    wid = lax.axis_index("s") * _NC + lax.axis_index("c")
    rbase = wid * _NCH
    pltpu.sync_copy(emb_hbm, tbl2.at[pl.ds(0, _S)])
    pltpu.sync_copy(emb_hbm, tbl2.at[pl.ds(_S, _S)])
    pltpu.sync_copy(idx_hbm.at[pl.ds(rbase, _NCH)], idx_all)

    def body(c4, carry):
        hs = []
        for kk in range(4):
            c = c4 * 4 + kk
            phase = lax.rem(c * _CHUNK, _S)
            hs.append(pltpu.async_copy(tbl2.at[pl.ds(phase, _CHUNK)],
                                       out_hbm.at[idx_all.at[c]], sem))
        for h in hs:
            h.wait()
        return carry

    lax.fori_loop(0, _NCH // 4, body, 0)


_sc_scatter = functools.partial(
    pl.kernel,
    mesh=plsc.VectorSubcoreMesh(core_axis_name="c", subcore_axis_name="s"),
    compiler_params=pltpu.CompilerParams(use_tc_tiling_on_sc=False),
    out_type=jax.ShapeDtypeStruct((_B * _S, _E), jnp.float32),
    scratch_types=[
        pltpu.VMEM((2 * _S, _E), jnp.float32),
        pltpu.VMEM((_NCH, _CHUNK), jnp.int32),
        pltpu.SemaphoreType.DMA,
    ],
)(_sc_scatter_body)


def kernel(ts, pos_embeddings):
    dest2 = _dest_rows(ts)
    out = _sc_scatter(pos_embeddings, dest2)
    return out.reshape(_B, _S, _E)


# speedup vs baseline: 1.5334x; 1.3365x over previous
import jax
import jax.numpy as jnp
from jax.experimental import pallas as pl

_B, _S, _E = 16384, 200, 64
_HL, _HH = 128, _S - 128
_R = 128


def _body(ts_ref, emb_ref, out_ref):
    ts = ts_ref[...]
    ib = jax.lax.bitcast_convert_type(ts, jnp.int32)
    ikey = ib ^ ((ib >> 31) & jnp.int32(0x7FFFFFFF))
    ikey = jnp.where(ts == 0.0, 0, ikey)
    kl, kh = ikey[:, :_HL], ikey[:, _HL:]
    al, bl = kl[:, :, None], kl[:, None, :]
    ah, bh = kh[:, :, None], kh[:, None, :]

    def _tri(n):
        k2 = jax.lax.broadcasted_iota(jnp.int32, (n, n), 1)
        j2 = jax.lax.broadcasted_iota(jnp.int32, (n, n), 0)
        return (k2 < j2).astype(jnp.int32)[None, :, :]

    cll = (bl < (al + _tri(_HL))).astype(jnp.float32)
    chh = (bh < (ah + _tri(_HH))).astype(jnp.float32)
    chl = (bl <= ah).astype(jnp.float32)
    rank_l = jnp.sum(cll, axis=2) + (_HH - jnp.sum(chl, axis=1))
    rank_h = jnp.sum(chh, axis=2) + jnp.sum(chl, axis=2)
    rank = jnp.concatenate([rank_l, rank_h], axis=1)
    i2 = jax.lax.broadcasted_iota(
        jnp.int32, (_S, _S), 0).astype(jnp.float32)[None, :, :]
    p = (i2 == rank[:, None, :]).astype(jnp.bfloat16)
    out = jnp.dot(p.reshape(_R * _S, _S), emb_ref[...].astype(jnp.bfloat16),
                  preferred_element_type=jnp.float32)
    out_ref[...] = out


def kernel(ts, pos_embeddings):
    out = pl.pallas_call(
        _body,
        grid=(_B // _R,),
        in_specs=[
            pl.BlockSpec((_R, _S), lambda i: (i, 0)),
            pl.BlockSpec((_S, _E), lambda i: (0, 0)),
        ],
        out_specs=pl.BlockSpec((_R * _S, _E), lambda i: (i, 0)),
        out_shape=jax.ShapeDtypeStruct((_B * _S, _E), jnp.float32),
    )(ts, pos_embeddings)
    return out.reshape(_B, _S, _E)


# speedup vs baseline: 2.3213x; 1.5138x over previous
import jax
import jax.numpy as jnp
from jax.experimental import pallas as pl

_B, _S, _E = 16384, 200, 64
_R = 128


def _body(ts_ref, emb_ref, out_ref):
    ts = ts_ref[...]
    ib = jax.lax.bitcast_convert_type(ts, jnp.int32)
    ikey = ib ^ ((ib >> 31) & jnp.int32(0x7FFFFFFF))
    ikey = jnp.where(ts == 0.0, 0, ikey)
    a = ikey[:, :, None]
    b = ikey[:, None, :]
    k2 = jax.lax.broadcasted_iota(jnp.int32, (_S, _S), 1)
    j2 = jax.lax.broadcasted_iota(jnp.int32, (_S, _S), 0)
    tri01 = (k2 < j2).astype(jnp.int32)[None, :, :]
    cmp = b < (a + tri01)
    rank = jnp.sum(cmp.astype(jnp.float32), axis=2)
    i2 = j2.astype(jnp.float32)[None, :, :]
    p = (i2 == rank[:, None, :]).astype(jnp.bfloat16)
    out = jnp.dot(p.reshape(_R * _S, _S), emb_ref[...].astype(jnp.bfloat16),
                  preferred_element_type=jnp.float32)
    out_ref[...] = out


def kernel(ts, pos_embeddings):
    out = pl.pallas_call(
        _body,
        grid=(_B // _R,),
        in_specs=[
            pl.BlockSpec((_R, _S), lambda i: (i, 0)),
            pl.BlockSpec((_S, _E), lambda i: (0, 0)),
        ],
        out_specs=pl.BlockSpec((_R * _S, _E), lambda i: (i, 0)),
        out_shape=jax.ShapeDtypeStruct((_B * _S, _E), jnp.float32),
    )(ts, pos_embeddings)
    return out.reshape(_B, _S, _E)
